# baseline (speedup 1.0000x reference)
import functools

import jax
import jax.numpy as jnp
from jax import lax
from jax.experimental import pallas as pl
from jax.experimental.pallas import tpu as pltpu

_BN_EPS = 1e-5


def _conv3x3_bf16(slab, w_ref, *, h, w, cin, cout):
    acc = jnp.zeros((h * w, cout), jnp.float32)
    for kh in range(3):
        for kw in range(3):
            xs = slab[kh:kh + h, kw:kw + w, :].reshape(h * w, cin)
            acc = acc + jnp.dot(xs, w_ref[kh * 3 + kw],
                                preferred_element_type=jnp.float32)
    return acc


def _stage1_kernel(x_ref, w1_ref, acc1_ref, psum_ref, psq_ref, *, h, w, cin, cout):
    slab = x_ref[0]
    acc = _conv3x3_bf16(slab, w1_ref, h=h, w=w, cin=cin, cout=cout)
    psum_ref[...] = jnp.sum(acc, axis=0).reshape(1, 1, cout)
    psq_ref[...] = jnp.sum(acc * acc, axis=0).reshape(1, 1, cout)
    acc1_ref[0, 1:h + 1, 1:w + 1, :] = acc.reshape(h, w, cout).astype(jnp.bfloat16)


def _stage2_kernel(acc1_ref, w2_ref, sc1_ref, sh1_ref,
                   acc2_ref, psum_ref, psq_ref, *, h, w, cout):
    a = acc1_ref[0].astype(jnp.float32)
    act = jnp.maximum(a * sc1_ref[0] + sh1_ref[0], 0.0)
    rows = lax.broadcasted_iota(jnp.int32, act.shape, 0)
    cols = lax.broadcasted_iota(jnp.int32, act.shape, 1)
    interior = (rows >= 1) & (rows <= h) & (cols >= 1) & (cols <= w)
    act = jnp.where(interior, act, 0.0).astype(jnp.bfloat16)
    acc = _conv3x3_bf16(act, w2_ref, h=h, w=w, cin=cout, cout=cout)
    psum_ref[...] = jnp.sum(acc, axis=0).reshape(1, 1, cout)
    psq_ref[...] = jnp.sum(acc * acc, axis=0).reshape(1, 1, cout)
    acc2_ref[0, 1:h + 1, 1:w + 1, :] = acc.reshape(h, w, cout).astype(jnp.bfloat16)


def _stage3_kernel(acc2_ref, x_ref, wrest_ref, sc2_ref, sh2_ref, bres_ref,
                   out_ref, *, h, w, cin, cout):
    a2 = acc2_ref[0, 1:h + 1, 1:w + 1, :].reshape(h * w, cout)
    a2t = jnp.transpose(a2).astype(jnp.float32)
    y = jnp.maximum(a2t * sc2_ref[...] + sh2_ref[...], 0.0)
    xs = x_ref[0].astype(jnp.bfloat16)
    res = jnp.dot(wrest_ref[...], xs,
                  preferred_element_type=jnp.float32) + bres_ref[...]
    out_ref[0] = jnp.maximum(y + res, 0.0)


def _bn_scale_shift(psum, psq, gamma, beta, m):
    s1 = jnp.sum(psum, axis=(0, 1))
    s2 = jnp.sum(psq, axis=(0, 1))
    mean = s1 / m
    var = jnp.maximum(s2 / m - mean * mean, 0.0)
    scale = gamma * lax.rsqrt(var + _BN_EPS)
    shift = beta - mean * scale
    return scale, shift


def kernel(x, w1, b1, g1, be1, w2, b2, g2, be2, wres, bres):
    N, Cin, H, W = x.shape
    Cout = w1.shape[-1]
    M = N * H * W
    Hp, Wp = H + 2, W + 2

    xf = x.reshape(N, Cin, H * W)
    x_pad = jnp.pad(jnp.transpose(x, (0, 2, 3, 1)).astype(jnp.bfloat16),
                    ((0, 0), (1, 1), (1, 1), (0, 0)))
    w1b = w1.reshape(9, Cin, Cout).astype(jnp.bfloat16)
    w2b = w2.reshape(9, Cout, Cout).astype(jnp.bfloat16)
    wrestb = wres.reshape(Cin, Cout).T.astype(jnp.bfloat16)

    cparams = pltpu.CompilerParams(dimension_semantics=("parallel",),
                                   vmem_limit_bytes=64 * 1024 * 1024)

    def const_spec(shape):
        return pl.BlockSpec(shape, lambda n: (0,) * len(shape))

    img = lambda c, dt: jax.ShapeDtypeStruct((N, Hp, Wp, c), dt)
    img_spec = lambda c: pl.BlockSpec((1, Hp, Wp, c), lambda n: (n, 0, 0, 0))
    x_spec = pl.BlockSpec((1, Cin, H * W), lambda n: (n, 0, 0))
    stat_spec = pl.BlockSpec((1, 1, Cout), lambda n: (n, 0, 0))
    stat_shape = jax.ShapeDtypeStruct((N, 1, Cout), jnp.float32)

    acc1, s1sum, s1sq = pl.pallas_call(
        functools.partial(_stage1_kernel, h=H, w=W, cin=Cin, cout=Cout),
        out_shape=(img(Cout, jnp.bfloat16), stat_shape, stat_shape),
        grid=(N,),
        in_specs=[img_spec(Cin), const_spec((9, Cin, Cout))],
        out_specs=(img_spec(Cout), stat_spec, stat_spec),
        compiler_params=cparams,
    )(x_pad, w1b)

    scale1, shift1 = _bn_scale_shift(s1sum, s1sq, g1, be1, M)

    acc2, s2sum, s2sq = pl.pallas_call(
        functools.partial(_stage2_kernel, h=H, w=W, cout=Cout),
        out_shape=(img(Cout, jnp.bfloat16), stat_shape, stat_shape),
        grid=(N,),
        in_specs=[img_spec(Cout), const_spec((9, Cout, Cout)),
                  const_spec((1, Cout)), const_spec((1, Cout))],
        out_specs=(img_spec(Cout), stat_spec, stat_spec),
        compiler_params=cparams,
    )(acc1, w2b, scale1, shift1)

    scale2, shift2 = _bn_scale_shift(s2sum, s2sq, g2, be2, M)

    out = pl.pallas_call(
        functools.partial(_stage3_kernel, h=H, w=W, cin=Cin, cout=Cout),
        out_shape=jax.ShapeDtypeStruct((N, Cout, H * W), jnp.float32),
        grid=(N,),
        in_specs=[img_spec(Cout), x_spec, const_spec((Cout, Cin)),
                  const_spec((Cout, 1)), const_spec((Cout, 1)),
                  const_spec((Cout, 1))],
        out_specs=pl.BlockSpec((1, Cout, H * W), lambda n: (n, 0, 0)),
        compiler_params=cparams,
    )(acc2, xf, wrestb, scale2.reshape(Cout, 1), shift2.reshape(Cout, 1),
      bres.reshape(Cout, 1))

    return out.reshape(N, Cout, H, W)


# speedup vs baseline: 1.4821x; 1.4821x over previous
import functools

import jax
import jax.numpy as jnp
from jax import lax
from jax.experimental import pallas as pl
from jax.experimental.pallas import tpu as pltpu

_BN_EPS = 1e-5


def _conv3x3_bf16(slab, w_ref, *, h, w, cin, cout):
    acc = jnp.zeros((h * w, cout), jnp.float32)
    for kh in range(3):
        for kw in range(3):
            xs = slab[kh:kh + h, kw:kw + w, :].reshape(h * w, cin)
            acc = acc + jnp.dot(xs, w_ref[kh * 3 + kw],
                                preferred_element_type=jnp.float32)
    return acc


def _stage1_kernel(x_ref, w1_ref, acc1_ref, psum_ref, psq_ref, slab,
                   *, h, w, cin, cout):
    slab[1:h + 1, 1:w + 1, :] = x_ref[0].astype(jnp.bfloat16)
    slab[0:1, :, :] = jnp.zeros((1, w + 2, cin), jnp.bfloat16)
    slab[h + 1:h + 2, :, :] = jnp.zeros((1, w + 2, cin), jnp.bfloat16)
    slab[:, 0:1, :] = jnp.zeros((h + 2, 1, cin), jnp.bfloat16)
    slab[:, w + 1:w + 2, :] = jnp.zeros((h + 2, 1, cin), jnp.bfloat16)
    acc = _conv3x3_bf16(slab[...], w1_ref, h=h, w=w, cin=cin, cout=cout)
    psum_ref[...] = jnp.sum(acc, axis=0).reshape(1, 1, cout)
    psq_ref[...] = jnp.sum(acc * acc, axis=0).reshape(1, 1, cout)
    acc1_ref[0, 1:h + 1, 1:w + 1, :] = acc.reshape(h, w, cout).astype(jnp.bfloat16)


def _stage2_kernel(acc1_ref, w2_ref, sc1_ref, sh1_ref,
                   acc2_ref, psum_ref, psq_ref, *, h, w, cout):
    a = acc1_ref[0].astype(jnp.float32)
    act = jnp.maximum(a * sc1_ref[0] + sh1_ref[0], 0.0)
    rows = lax.broadcasted_iota(jnp.int32, act.shape, 0)
    cols = lax.broadcasted_iota(jnp.int32, act.shape, 1)
    interior = (rows >= 1) & (rows <= h) & (cols >= 1) & (cols <= w)
    act = jnp.where(interior, act, 0.0).astype(jnp.bfloat16)
    acc = _conv3x3_bf16(act, w2_ref, h=h, w=w, cin=cout, cout=cout)
    psum_ref[...] = jnp.sum(acc, axis=0).reshape(1, 1, cout)
    psq_ref[...] = jnp.sum(acc * acc, axis=0).reshape(1, 1, cout)
    acc2_ref[0, 1:h + 1, 1:w + 1, :] = acc.reshape(h, w, cout).astype(jnp.bfloat16)


def _stage3_kernel(acc2_ref, x_ref, wres_ref, sc2_ref, sh2_ref, bres_ref,
                   out_ref, *, h, w, cin, cout):
    a2 = acc2_ref[0, 1:h + 1, 1:w + 1, :].astype(jnp.float32)
    y = jnp.maximum(a2 * sc2_ref[0] + sh2_ref[0], 0.0)
    xs = x_ref[0].reshape(h * w, cin).astype(jnp.bfloat16)
    res = jnp.dot(xs, wres_ref[...],
                  preferred_element_type=jnp.float32) + bres_ref[0]
    out_ref[0] = jnp.maximum(y + res.reshape(h, w, cout), 0.0)


def _bn_scale_shift(psum, psq, gamma, beta, m):
    s1 = jnp.sum(psum, axis=(0, 1))
    s2 = jnp.sum(psq, axis=(0, 1))
    mean = s1 / m
    var = jnp.maximum(s2 / m - mean * mean, 0.0)
    scale = gamma * lax.rsqrt(var + _BN_EPS)
    shift = beta - mean * scale
    return scale, shift


def kernel(x, w1, b1, g1, be1, w2, b2, g2, be2, wres, bres):
    N, Cin, H, W = x.shape
    Cout = w1.shape[-1]
    M = N * H * W
    Hp, Wp = H + 2, W + 2

    xt = jnp.transpose(x, (0, 2, 3, 1))
    w1b = w1.reshape(9, Cin, Cout).astype(jnp.bfloat16)
    w2b = w2.reshape(9, Cout, Cout).astype(jnp.bfloat16)
    wresb = wres.reshape(Cin, Cout).astype(jnp.bfloat16)

    cparams = pltpu.CompilerParams(dimension_semantics=("parallel",),
                                   vmem_limit_bytes=64 * 1024 * 1024)

    def const_spec(shape):
        return pl.BlockSpec(shape, lambda n: (0,) * len(shape))

    img = lambda c, dt: jax.ShapeDtypeStruct((N, Hp, Wp, c), dt)
    img_spec = lambda c: pl.BlockSpec((1, Hp, Wp, c), lambda n: (n, 0, 0, 0))
    x_spec = pl.BlockSpec((1, H, W, Cin), lambda n: (n, 0, 0, 0))
    stat_spec = pl.BlockSpec((1, 1, Cout), lambda n: (n, 0, 0))
    stat_shape = jax.ShapeDtypeStruct((N, 1, Cout), jnp.float32)

    acc1, s1sum, s1sq = pl.pallas_call(
        functools.partial(_stage1_kernel, h=H, w=W, cin=Cin, cout=Cout),
        out_shape=(img(Cout, jnp.bfloat16), stat_shape, stat_shape),
        grid=(N,),
        in_specs=[x_spec, const_spec((9, Cin, Cout))],
        out_specs=(img_spec(Cout), stat_spec, stat_spec),
        scratch_shapes=[pltpu.VMEM((Hp, Wp, Cin), jnp.bfloat16)],
        compiler_params=cparams,
    )(xt, w1b)

    scale1, shift1 = _bn_scale_shift(s1sum, s1sq, g1, be1, M)

    acc2, s2sum, s2sq = pl.pallas_call(
        functools.partial(_stage2_kernel, h=H, w=W, cout=Cout),
        out_shape=(img(Cout, jnp.bfloat16), stat_shape, stat_shape),
        grid=(N,),
        in_specs=[img_spec(Cout), const_spec((9, Cout, Cout)),
                  const_spec((1, Cout)), const_spec((1, Cout))],
        out_specs=(img_spec(Cout), stat_spec, stat_spec),
        compiler_params=cparams,
    )(acc1, w2b, scale1, shift1)

    scale2, shift2 = _bn_scale_shift(s2sum, s2sq, g2, be2, M)

    out = pl.pallas_call(
        functools.partial(_stage3_kernel, h=H, w=W, cin=Cin, cout=Cout),
        out_shape=jax.ShapeDtypeStruct((N, H, W, Cout), jnp.float32),
        grid=(N,),
        in_specs=[img_spec(Cout), x_spec, const_spec((Cin, Cout)),
                  const_spec((1, Cout)), const_spec((1, Cout)),
                  const_spec((1, Cout))],
        out_specs=pl.BlockSpec((1, H, W, Cout), lambda n: (n, 0, 0, 0)),
        compiler_params=cparams,
    )(acc2, xt, wresb, scale2, shift2, bres)

    return jnp.transpose(out, (0, 3, 1, 2))


# speedup vs baseline: 1.4990x; 1.0114x over previous
import functools

import jax
import jax.numpy as jnp
from jax import lax
from jax.experimental import pallas as pl
from jax.experimental.pallas import tpu as pltpu

_BN_EPS = 1e-5


def _conv3x3_bf16(slab, w_ref, *, h, w, cin, cout):
    acc = jnp.zeros((h * w, cout), jnp.float32)
    for kh in range(3):
        for kw in range(3):
            xs = slab[kh:kh + h, kw:kw + w, :].reshape(h * w, cin)
            acc = acc + jnp.dot(xs, w_ref[kh * 3 + kw],
                                preferred_element_type=jnp.float32)
    return acc


def _bn_fold(psum_ref, psq_ref, gamma_ref, beta_ref, m):
    s1 = jnp.sum(psum_ref[...], axis=(0, 1))
    s2 = jnp.sum(psq_ref[...], axis=(0, 1))
    mean = s1 / m
    var = jnp.maximum(s2 / m - mean * mean, 0.0)
    scale = gamma_ref[0] * lax.rsqrt(var + _BN_EPS)
    shift = beta_ref[0] - mean * scale
    return scale, shift


def _stage1_kernel(x_ref, w1_ref, acc1_ref, psum_ref, psq_ref, slab,
                   *, h, w, cin, cout):
    slab[1:h + 1, 1:w + 1, :] = x_ref[0].astype(jnp.bfloat16)
    slab[0:1, :, :] = jnp.zeros((1, w + 2, cin), jnp.bfloat16)
    slab[h + 1:h + 2, :, :] = jnp.zeros((1, w + 2, cin), jnp.bfloat16)
    slab[:, 0:1, :] = jnp.zeros((h + 2, 1, cin), jnp.bfloat16)
    slab[:, w + 1:w + 2, :] = jnp.zeros((h + 2, 1, cin), jnp.bfloat16)
    acc = _conv3x3_bf16(slab[...], w1_ref, h=h, w=w, cin=cin, cout=cout)
    psum_ref[...] = jnp.sum(acc, axis=0).reshape(1, 1, cout)
    psq_ref[...] = jnp.sum(acc * acc, axis=0).reshape(1, 1, cout)
    acc1_ref[0, 1:h + 1, 1:w + 1, :] = acc.reshape(h, w, cout).astype(jnp.bfloat16)


def _stage2_kernel(acc1_ref, w2_ref, s1sum_ref, s1sq_ref, g1_ref, be1_ref,
                   acc2_ref, psum_ref, psq_ref, *, h, w, cout, m):
    sc1, sh1 = _bn_fold(s1sum_ref, s1sq_ref, g1_ref, be1_ref, m)
    a = acc1_ref[0].astype(jnp.float32)
    act = jnp.maximum(a * sc1 + sh1, 0.0)
    rows = lax.broadcasted_iota(jnp.int32, act.shape, 0)
    cols = lax.broadcasted_iota(jnp.int32, act.shape, 1)
    interior = (rows >= 1) & (rows <= h) & (cols >= 1) & (cols <= w)
    act = jnp.where(interior, act, 0.0).astype(jnp.bfloat16)
    acc = _conv3x3_bf16(act, w2_ref, h=h, w=w, cin=cout, cout=cout)
    psum_ref[...] = jnp.sum(acc, axis=0).reshape(1, 1, cout)
    psq_ref[...] = jnp.sum(acc * acc, axis=0).reshape(1, 1, cout)
    acc2_ref[0, 1:h + 1, 1:w + 1, :] = acc.reshape(h, w, cout).astype(jnp.bfloat16)


def _stage3_kernel(acc2_ref, x_ref, wres_ref, s2sum_ref, s2sq_ref, g2_ref,
                   be2_ref, bres_ref, out_ref, *, h, w, cin, cout, m):
    sc2, sh2 = _bn_fold(s2sum_ref, s2sq_ref, g2_ref, be2_ref, m)
    a2 = acc2_ref[0, 1:h + 1, 1:w + 1, :].astype(jnp.float32)
    y = jnp.maximum(a2 * sc2 + sh2, 0.0)
    xs = x_ref[0].reshape(h * w, cin).astype(jnp.bfloat16)
    res = jnp.dot(xs, wres_ref[...],
                  preferred_element_type=jnp.float32) + bres_ref[0]
    out_ref[0] = jnp.maximum(y + res.reshape(h, w, cout), 0.0)


def kernel(x, w1, b1, g1, be1, w2, b2, g2, be2, wres, bres):
    N, Cin, H, W = x.shape
    Cout = w1.shape[-1]
    M = N * H * W
    Hp, Wp = H + 2, W + 2

    xt = jnp.transpose(x, (0, 2, 3, 1))
    w1b = w1.reshape(9, Cin, Cout).astype(jnp.bfloat16)
    w2b = w2.reshape(9, Cout, Cout).astype(jnp.bfloat16)
    wresb = wres.reshape(Cin, Cout).astype(jnp.bfloat16)

    cparams = pltpu.CompilerParams(dimension_semantics=("parallel",),
                                   vmem_limit_bytes=64 * 1024 * 1024)

    def const_spec(shape):
        return pl.BlockSpec(shape, lambda n: (0,) * len(shape))

    img = lambda c, dt: jax.ShapeDtypeStruct((N, Hp, Wp, c), dt)
    img_spec = lambda c: pl.BlockSpec((1, Hp, Wp, c), lambda n: (n, 0, 0, 0))
    x_spec = pl.BlockSpec((1, H, W, Cin), lambda n: (n, 0, 0, 0))
    stat_spec = pl.BlockSpec((1, 1, Cout), lambda n: (n, 0, 0))
    stat_shape = jax.ShapeDtypeStruct((N, 1, Cout), jnp.float32)

    acc1, s1sum, s1sq = pl.pallas_call(
        functools.partial(_stage1_kernel, h=H, w=W, cin=Cin, cout=Cout),
        out_shape=(img(Cout, jnp.bfloat16), stat_shape, stat_shape),
        grid=(N,),
        in_specs=[x_spec, const_spec((9, Cin, Cout))],
        out_specs=(img_spec(Cout), stat_spec, stat_spec),
        scratch_shapes=[pltpu.VMEM((Hp, Wp, Cin), jnp.bfloat16)],
        compiler_params=cparams,
    )(xt, w1b)

    acc2, s2sum, s2sq = pl.pallas_call(
        functools.partial(_stage2_kernel, h=H, w=W, cout=Cout, m=float(M)),
        out_shape=(img(Cout, jnp.bfloat16), stat_shape, stat_shape),
        grid=(N,),
        in_specs=[img_spec(Cout), const_spec((9, Cout, Cout)),
                  const_spec((N, 1, Cout)), const_spec((N, 1, Cout)),
                  const_spec((1, Cout)), const_spec((1, Cout))],
        out_specs=(img_spec(Cout), stat_spec, stat_spec),
        compiler_params=cparams,
    )(acc1, w2b, s1sum, s1sq, g1, be1)

    out = pl.pallas_call(
        functools.partial(_stage3_kernel, h=H, w=W, cin=Cin, cout=Cout,
                          m=float(M)),
        out_shape=jax.ShapeDtypeStruct((N, H, W, Cout), jnp.float32),
        grid=(N,),
        in_specs=[img_spec(Cout), x_spec, const_spec((Cin, Cout)),
                  const_spec((N, 1, Cout)), const_spec((N, 1, Cout)),
                  const_spec((1, Cout)), const_spec((1, Cout)),
                  const_spec((1, Cout))],
        out_specs=pl.BlockSpec((1, H, W, Cout), lambda n: (n, 0, 0, 0)),
        compiler_params=cparams,
    )(acc2, xt, wresb, s2sum, s2sq, g2, be2, bres)

    return jnp.transpose(out, (0, 3, 1, 2))


# speedup vs baseline: 1.5200x; 1.0140x over previous
import functools

import jax
import jax.numpy as jnp
from jax import lax
from jax.experimental import pallas as pl
from jax.experimental.pallas import tpu as pltpu

_BN_EPS = 1e-5


def _conv3x3_bf16(slab, w_ref, *, h, w, cin, cout):
    acc = jnp.zeros((h, w, cout), jnp.float32)
    for kh in range(3):
        for kw in range(3):
            xs = slab[kh:kh + h, kw:kw + w, :]
            acc = acc + lax.dot_general(
                xs, w_ref[kh * 3 + kw], (((2,), (0,)), ((), ())),
                preferred_element_type=jnp.float32)
    return acc


def _bn_fold(psum_ref, psq_ref, gamma_ref, beta_ref, m):
    s1 = jnp.sum(psum_ref[...], axis=(0, 1))
    s2 = jnp.sum(psq_ref[...], axis=(0, 1))
    mean = s1 / m
    var = jnp.maximum(s2 / m - mean * mean, 0.0)
    scale = gamma_ref[0] * lax.rsqrt(var + _BN_EPS)
    shift = beta_ref[0] - mean * scale
    return scale, shift


def _stage1_kernel(x_ref, w1_ref, acc1_ref, psum_ref, psq_ref, slab,
                   *, h, w, cin, cout):
    slab[1:h + 1, 1:w + 1, :] = x_ref[0].astype(jnp.bfloat16)
    slab[0:1, :, :] = jnp.zeros((1, w + 2, cin), jnp.bfloat16)
    slab[h + 1:h + 2, :, :] = jnp.zeros((1, w + 2, cin), jnp.bfloat16)
    slab[:, 0:1, :] = jnp.zeros((h + 2, 1, cin), jnp.bfloat16)
    slab[:, w + 1:w + 2, :] = jnp.zeros((h + 2, 1, cin), jnp.bfloat16)
    acc = _conv3x3_bf16(slab[...], w1_ref, h=h, w=w, cin=cin, cout=cout)
    psum_ref[...] = jnp.sum(acc, axis=(0, 1)).reshape(1, 1, cout)
    psq_ref[...] = jnp.sum(acc * acc, axis=(0, 1)).reshape(1, 1, cout)
    acc1_ref[0, 1:h + 1, 1:w + 1, :] = acc.astype(jnp.bfloat16)


def _stage2_kernel(acc1_ref, w2_ref, s1sum_ref, s1sq_ref, g1_ref, be1_ref,
                   acc2_ref, psum_ref, psq_ref, *, h, w, cout, m):
    sc1, sh1 = _bn_fold(s1sum_ref, s1sq_ref, g1_ref, be1_ref, m)
    a = acc1_ref[0].astype(jnp.float32)
    act = jnp.maximum(a * sc1 + sh1, 0.0)
    rows = lax.broadcasted_iota(jnp.int32, act.shape, 0)
    cols = lax.broadcasted_iota(jnp.int32, act.shape, 1)
    interior = (rows >= 1) & (rows <= h) & (cols >= 1) & (cols <= w)
    act = jnp.where(interior, act, 0.0).astype(jnp.bfloat16)
    acc = _conv3x3_bf16(act, w2_ref, h=h, w=w, cin=cout, cout=cout)
    psum_ref[...] = jnp.sum(acc, axis=(0, 1)).reshape(1, 1, cout)
    psq_ref[...] = jnp.sum(acc * acc, axis=(0, 1)).reshape(1, 1, cout)
    acc2_ref[0, 1:h + 1, 1:w + 1, :] = acc.astype(jnp.bfloat16)


def _stage3_kernel(acc2_ref, x_ref, wres_ref, s2sum_ref, s2sq_ref, g2_ref,
                   be2_ref, bres_ref, out_ref, *, h, w, cin, cout, m):
    sc2, sh2 = _bn_fold(s2sum_ref, s2sq_ref, g2_ref, be2_ref, m)
    a2 = acc2_ref[0, 1:h + 1, 1:w + 1, :].astype(jnp.float32)
    y = jnp.maximum(a2 * sc2 + sh2, 0.0)
    xs = x_ref[0].reshape(h * w, cin).astype(jnp.bfloat16)
    res = jnp.dot(xs, wres_ref[...],
                  preferred_element_type=jnp.float32) + bres_ref[0]
    out_ref[0] = jnp.maximum(y + res.reshape(h, w, cout), 0.0)


def kernel(x, w1, b1, g1, be1, w2, b2, g2, be2, wres, bres):
    N, Cin, H, W = x.shape
    Cout = w1.shape[-1]
    M = N * H * W
    Hp, Wp = H + 2, W + 2

    xt = jnp.transpose(x, (0, 2, 3, 1))
    w1b = w1.reshape(9, Cin, Cout).astype(jnp.bfloat16)
    w2b = w2.reshape(9, Cout, Cout).astype(jnp.bfloat16)
    wresb = wres.reshape(Cin, Cout).astype(jnp.bfloat16)

    cparams = pltpu.CompilerParams(dimension_semantics=("parallel",),
                                   vmem_limit_bytes=64 * 1024 * 1024)

    def const_spec(shape):
        return pl.BlockSpec(shape, lambda n: (0,) * len(shape))

    img = lambda c, dt: jax.ShapeDtypeStruct((N, Hp, Wp, c), dt)
    img_spec = lambda c: pl.BlockSpec((1, Hp, Wp, c), lambda n: (n, 0, 0, 0))
    x_spec = pl.BlockSpec((1, H, W, Cin), lambda n: (n, 0, 0, 0))
    stat_spec = pl.BlockSpec((1, 1, Cout), lambda n: (n, 0, 0))
    stat_shape = jax.ShapeDtypeStruct((N, 1, Cout), jnp.float32)

    acc1, s1sum, s1sq = pl.pallas_call(
        functools.partial(_stage1_kernel, h=H, w=W, cin=Cin, cout=Cout),
        out_shape=(img(Cout, jnp.bfloat16), stat_shape, stat_shape),
        grid=(N,),
        in_specs=[x_spec, const_spec((9, Cin, Cout))],
        out_specs=(img_spec(Cout), stat_spec, stat_spec),
        scratch_shapes=[pltpu.VMEM((Hp, Wp, Cin), jnp.bfloat16)],
        compiler_params=cparams,
    )(xt, w1b)

    acc2, s2sum, s2sq = pl.pallas_call(
        functools.partial(_stage2_kernel, h=H, w=W, cout=Cout, m=float(M)),
        out_shape=(img(Cout, jnp.bfloat16), stat_shape, stat_shape),
        grid=(N,),
        in_specs=[img_spec(Cout), const_spec((9, Cout, Cout)),
                  const_spec((N, 1, Cout)), const_spec((N, 1, Cout)),
                  const_spec((1, Cout)), const_spec((1, Cout))],
        out_specs=(img_spec(Cout), stat_spec, stat_spec),
        compiler_params=cparams,
    )(acc1, w2b, s1sum, s1sq, g1, be1)

    out = pl.pallas_call(
        functools.partial(_stage3_kernel, h=H, w=W, cin=Cin, cout=Cout,
                          m=float(M)),
        out_shape=jax.ShapeDtypeStruct((N, H, W, Cout), jnp.float32),
        grid=(N,),
        in_specs=[img_spec(Cout), x_spec, const_spec((Cin, Cout)),
                  const_spec((N, 1, Cout)), const_spec((N, 1, Cout)),
                  const_spec((1, Cout)), const_spec((1, Cout)),
                  const_spec((1, Cout))],
        out_specs=pl.BlockSpec((1, H, W, Cout), lambda n: (n, 0, 0, 0)),
        compiler_params=cparams,
    )(acc2, xt, wresb, s2sum, s2sq, g2, be2, bres)

    return jnp.transpose(out, (0, 3, 1, 2))
